# baseline (speedup 1.0000x reference)
import functools

import jax
import jax.numpy as jnp
from jax import lax
from jax.experimental import pallas as pl
from jax.experimental.pallas import tpu as pltpu
from jax.experimental.pallas import tpu_sc as plsc

_B = 16384
_D = 64
_V = 1000000
_L = 16
_NC = 2
_NS = 16
_NW = _NC * _NS
_NP = 1954
_NU = 62
_PCAP = 16896
_STG = 144


def _pc(mask):
    return plsc.all_reduce_population_count(mask)[0]


def _kernel_a_body(tableT, labels, packed, bidx, meta,
                   lblstage, u_pack, s_pack, blk, pack2d, bstage,
                   metastage, sent, smem_off, sem):
    w = lax.axis_index("s") * _NC + lax.axis_index("c")
    iota = lax.iota(jnp.int32, _L)

    def quad_src(u):
        offl = pl.multiple_of((32 * u + w) * 512, 128)
        return tableT.at[:, pl.ds(offl, 512)]

    pltpu.async_copy(quad_src(jnp.int32(0)), blk.at[0], sem)

    def stage_loop(st, carry):
        pltpu.sync_copy(
            labels.at[pl.ds(pl.multiple_of(st * 2048, 2048), 2048)], lblstage)

        def chunk_loop(ci, c2):
            n, ltv = c2
            lb = lblstage[pl.ds(ci * _L, _L)]
            owner = lax.bitwise_and(lax.shift_right_logical(lb, 9), 31)
            mine = owner == w
            ltv = ltv + jnp.where(owner < w, 1, 0)
            bpos = iota + (st * 2048 + ci * _L)
            entry = lax.bitwise_or(
                lax.shift_left(lax.shift_right_logical(lb, 14), 23),
                lax.bitwise_or(lax.shift_left(bpos, 9),
                               lax.bitwise_and(lb, 511)))
            plsc.store_compressed(u_pack.at[pl.ds(n, _L)], entry, mask=mine)
            return n + _pc(mine), ltv

        return lax.fori_loop(0, 2048 // _L, chunk_loop, carry, unroll=4)

    n_mine, n_ltv = lax.fori_loop(
        0, _B // 2048, stage_loop,
        (jnp.int32(0), jnp.zeros((_L,), jnp.int32)), unroll=False)
    n_lt = plsc.cumsum(n_ltv)[_L - 1]

    base = lax.bitwise_and(n_lt + 7, ~7) + 8 * w
    region_end = jnp.where(
        w == _NW - 1,
        jnp.int32(_PCAP),
        lax.bitwise_and(n_lt + n_mine + 7, ~7) + 8 * (w + 1),
    )

    nch = lax.shift_right_logical(n_mine + _L - 1, 4)
    off = jnp.int32(0)
    lvl1 = [jnp.int32(0)] * 9
    for s1 in range(8):
        lvl1[s1] = off

        def l1_loop(ci, acc, s1=s1):
            e = u_pack[pl.ds(ci * _L, _L)]
            valid = (iota + ci * _L) < n_mine
            msk = jnp.logical_and(valid, lax.shift_right_logical(e, 26) == s1)
            plsc.store_compressed(s_pack.at[pl.ds(acc, _L)], e, mask=msk)
            return acc + _pc(msk)

        off = lax.fori_loop(0, nch, l1_loop, off, unroll=False)
    lvl1[8] = off

    off = jnp.int32(0)
    for s1 in range(8):
        b0 = lvl1[s1]
        b1 = lvl1[s1 + 1]
        c0b = lax.shift_right_logical(b0, 4)
        c1b = lax.shift_right_logical(b1 + _L - 1, 4)
        for s2 in range(8):
            u = s1 * 8 + s2
            if u >= _NU:
                continue
            smem_off[u] = off

            def l2_loop(ci, acc, u=u, b0=b0, b1=b1):
                e = s_pack[pl.ds(ci * _L, _L)]
                gpos = iota + ci * _L
                valid = jnp.logical_and(gpos >= b0, gpos < b1)
                msk = jnp.logical_and(
                    valid, lax.shift_right_logical(e, 23) == u)
                plsc.store_compressed(u_pack.at[pl.ds(acc, _L)], e, mask=msk)
                return acc + _pc(msk)

            off = lax.fori_loop(c0b, c1b, l2_loop, off, unroll=False)
    smem_off[_NU] = off

    for jj in range(_STG // _L):
        bstage[pl.ds(jj * _L, _L)] = jnp.full((_L,), -1, jnp.int32)

    def extract_chunks(start, end, parity, nrow, nflush):
        c0 = lax.shift_right_logical(start, 4)
        c1 = lax.shift_right_logical(end + _L - 1, 4)

        def chunk(ci, c2):
            nrow, nflush = c2
            e = u_pack[pl.ds(ci * _L, _L)]
            gpos = iota + ci * _L
            msk = jnp.logical_and(gpos >= start, gpos < end)
            lane = lax.bitwise_and(e, 511)
            bb = lax.bitwise_and(lax.shift_right_logical(e, 9), 16383)
            pvec = jnp.broadcast_to(parity, (_L,))
            pos = nrow + plsc.cumsum(jnp.where(msk, 1, 0)) - 1
            for cc in range(_D):
                vals = plsc.load_gather(
                    blk, [pvec, jnp.full((_L,), cc, jnp.int32), lane])
                plsc.store_scatter(pack2d,
                                   [pos, jnp.full((_L,), cc, jnp.int32)],
                                   vals, mask=msk)
            plsc.store_scatter(bstage, [pos], bb, mask=msk)
            nrow = nrow + _pc(msk)

            do_flush = nrow >= 128

            @pl.when(do_flush)
            def _():
                dst = pl.multiple_of(base + 128 * nflush, 8)
                pltpu.sync_copy(pack2d.at[pl.ds(0, 128)],
                                packed.at[pl.ds(dst, 128)])
                pltpu.sync_copy(bstage.at[pl.ds(0, 128)],
                                bidx.at[pl.ds(dst, 128)])
                for r in range(_L):
                    for c4 in range(_D // _L):
                        pack2d[r, pl.ds(c4 * _L, _L)] = (
                            pack2d[128 + r, pl.ds(c4 * _L, _L)])
                bstage[pl.ds(0, _L)] = bstage[pl.ds(128, _L)]
                for jj in range(1, _STG // _L):
                    bstage[pl.ds(jj * _L, _L)] = jnp.full((_L,), -1, jnp.int32)

            nrow = jnp.where(do_flush, nrow - 128, nrow)
            nflush = jnp.where(do_flush, nflush + 1, nflush)
            return nrow, nflush

        return lax.fori_loop(c0, c1, chunk, (nrow, nflush), unroll=False)

    def group(u, carry):
        nrow, nflush = carry
        parity = lax.bitwise_and(u, 1)
        pltpu.make_async_copy(quad_src(u), blk.at[parity], sem).wait()

        @pl.when(u < _NU - 2)
        def _():
            pltpu.async_copy(quad_src(u + 1),
                             blk.at[lax.bitwise_and(u + 1, 1)], sem)

        start = smem_off[u]
        end = smem_off[u + 1]
        return extract_chunks(start, end, parity, nrow, nflush)

    nrow, nflush = lax.fori_loop(0, _NU - 1, group,
                                 (jnp.int32(0), jnp.int32(0)), unroll=False)

    t_start = smem_off[_NU - 1]
    t_end = smem_off[_NU]
    has_tail = t_end > t_start

    @pl.when(jnp.logical_and(has_tail, w == 0))
    def _():
        pltpu.sync_copy(quad_src(jnp.int32(_NU - 1)), blk.at[1])

    @pl.when(jnp.logical_and(has_tail, w == 1))
    def _():
        offl = pl.multiple_of((32 * (_NU - 1) + w) * 512, 128)
        pltpu.sync_copy(tableT.at[:, pl.ds(offl, 128)],
                        blk.at[1, :, pl.ds(0, 128)])

    nrow, nflush = extract_chunks(t_start, t_end, jnp.int32(1), nrow, nflush)

    for jj in range(_STG // _L):
        v = bstage[pl.ds(jj * _L, _L)]
        keep = (iota + jj * _L) < nrow
        bstage[pl.ds(jj * _L, _L)] = jnp.where(keep, v, -1)

    nsub = lax.shift_right_logical(nrow + 7, 3)

    def tail_flush(jj, _):
        dst = pl.multiple_of(base + 128 * nflush + 8 * jj, 8)
        pltpu.sync_copy(pack2d.at[pl.ds(8 * jj, 8)], packed.at[pl.ds(dst, 8)])
        pltpu.sync_copy(bstage.at[pl.ds(8 * jj, 8)], bidx.at[pl.ds(dst, 8)])
        return 0

    lax.fori_loop(0, nsub, tail_flush, 0, unroll=False)

    tail_end = base + 128 * nflush + 8 * nsub
    sent[pl.ds(0, _L)] = jnp.full((_L,), -1, jnp.int32)
    ngap = lax.shift_right_logical(region_end - tail_end, 3)

    def gap_fill(jj, _):
        pltpu.sync_copy(
            sent.at[pl.ds(0, 8)],
            bidx.at[pl.ds(pl.multiple_of(tail_end + 8 * jj, 8), 8)])
        return 0

    lax.fori_loop(0, ngap, gap_fill, 0, unroll=False)

    mv = jnp.where(iota == 0, base, jnp.where(iota == 1, n_mine, 0))
    metastage[pl.ds(0, _L)] = mv
    pltpu.sync_copy(metastage,
                    meta.at[pl.ds(pl.multiple_of(_L * w, _L), _L)])


def _kernel_b_body(packed, bidx, meta, out, rows_v, bvec_v, meta_v, sem):
    w = lax.axis_index("s") * _NC + lax.axis_index("c")
    pltpu.sync_copy(meta.at[pl.ds(pl.multiple_of(_L * w, _L), _L)], meta_v)
    mv = meta_v[pl.ds(0, _L)]
    base = mv[0]
    cnt = mv[1]
    trips = lax.shift_right_logical(cnt + 127, 7)

    def trip(k, _):
        src = pl.multiple_of(base + 128 * k, 8)
        pltpu.sync_copy(packed.at[pl.ds(src, 128)], rows_v)
        pltpu.sync_copy(bidx.at[pl.ds(src, 128)], bvec_v)
        pltpu.async_copy(
            rows_v, out.at[plsc.Indices(bvec_v, ignored_value=-1)], sem
        ).wait()
        return 0

    lax.fori_loop(0, trips, trip, 0, unroll=False)


def _make_kernels():
    mesh = plsc.VectorSubcoreMesh(core_axis_name="c", subcore_axis_name="s")
    kern_a = functools.partial(
        pl.kernel,
        mesh=mesh,
        out_type=(
            jax.ShapeDtypeStruct((_PCAP, _D), jnp.float32),
            jax.ShapeDtypeStruct((_PCAP,), jnp.int32),
            jax.ShapeDtypeStruct((_L * _NW,), jnp.int32),
        ),
        scratch_types=[
            pltpu.VMEM((2048,), jnp.int32),
            pltpu.VMEM((_B + _L * 2,), jnp.int32),
            pltpu.VMEM((_B + _L * 2,), jnp.int32),
            pltpu.VMEM((2, _D, 512), jnp.float32),
            pltpu.VMEM((_STG, _D), jnp.float32),
            pltpu.VMEM((_STG,), jnp.int32),
            pltpu.VMEM((_L,), jnp.int32),
            pltpu.VMEM((_L,), jnp.int32),
            pltpu.SMEM((_NU + 2,), jnp.int32),
            pltpu.SemaphoreType.DMA,
        ],
        compiler_params=pltpu.CompilerParams(needs_layout_passes=False),
    )(_kernel_a_body)

    kern_b = functools.partial(
        pl.kernel,
        mesh=mesh,
        out_type=jax.ShapeDtypeStruct((_B, _D), jnp.float32),
        scratch_types=[
            pltpu.VMEM((128, _D), jnp.float32),
            pltpu.VMEM((128,), jnp.int32),
            pltpu.VMEM((_L,), jnp.int32),
            pltpu.SemaphoreType.DMA,
        ],
        compiler_params=pltpu.CompilerParams(
            use_tc_tiling_on_sc=False, needs_layout_passes=False),
    )(_kernel_b_body)
    return kern_a, kern_b


def kernel(labels, embedding_table):
    kern_a, kern_b = _make_kernels()
    lab = labels.astype(jnp.int32)
    tT = jnp.swapaxes(embedding_table, 0, 1)
    packed, bidx, meta = kern_a(tT, lab)
    return kern_b(packed, bidx, meta)


# speedup vs baseline: 1.1472x; 1.1472x over previous
import functools

import jax
import jax.numpy as jnp
from jax import lax
from jax.experimental import pallas as pl
from jax.experimental.pallas import tpu as pltpu
from jax.experimental.pallas import tpu_sc as plsc

_B = 16384
_D = 64
_V = 1000000
_L = 16
_NC = 2
_NS = 16
_NW = _NC * _NS
_NP = 1954
_NU = 62
_STG = 144


def _pc(mask):
    return plsc.all_reduce_population_count(mask)[0]


def _kernel_body(tableT, labels, out,
                 lblstage, u_pack, s_pack, blk, pack2d, bstage, bvec,
                 smem_off, sem, sem2):
    w = lax.axis_index("s") * _NC + lax.axis_index("c")
    iota = lax.iota(jnp.int32, _L)

    def quad_src(u):
        offl = pl.multiple_of((32 * u + w) * 512, 128)
        return tableT.at[:, pl.ds(offl, 512)]

    pltpu.async_copy(quad_src(jnp.int32(0)), blk.at[0], sem)

    def stage_loop(st, carry):
        pltpu.sync_copy(
            labels.at[pl.ds(pl.multiple_of(st * 2048, 2048), 2048)], lblstage)

        def chunk_loop(ci, n):
            lb = lblstage[pl.ds(ci * _L, _L)]
            owner = lax.bitwise_and(lax.shift_right_logical(lb, 9), 31)
            mine = owner == w
            bpos = iota + (st * 2048 + ci * _L)
            entry = lax.bitwise_or(
                lax.shift_left(lax.shift_right_logical(lb, 14), 23),
                lax.bitwise_or(lax.shift_left(bpos, 9),
                               lax.bitwise_and(lb, 511)))
            plsc.store_compressed(u_pack.at[pl.ds(n, _L)], entry, mask=mine)
            return n + _pc(mine)

        return lax.fori_loop(0, 2048 // _L, chunk_loop, carry, unroll=4)

    n_mine = lax.fori_loop(0, _B // 2048, stage_loop, jnp.int32(0),
                           unroll=False)

    nch = lax.shift_right_logical(n_mine + _L - 1, 4)
    off = jnp.int32(0)
    lvl1 = [jnp.int32(0)] * 9
    for s1 in range(8):
        lvl1[s1] = off

        def l1_loop(ci, acc, s1=s1):
            e = u_pack[pl.ds(ci * _L, _L)]
            valid = (iota + ci * _L) < n_mine
            msk = jnp.logical_and(valid, lax.shift_right_logical(e, 26) == s1)
            plsc.store_compressed(s_pack.at[pl.ds(acc, _L)], e, mask=msk)
            return acc + _pc(msk)

        off = lax.fori_loop(0, nch, l1_loop, off, unroll=False)
    lvl1[8] = off

    off = jnp.int32(0)
    for s1 in range(8):
        b0 = lvl1[s1]
        b1 = lvl1[s1 + 1]
        c0b = lax.shift_right_logical(b0, 4)
        c1b = lax.shift_right_logical(b1 + _L - 1, 4)
        for s2 in range(8):
            u = s1 * 8 + s2
            if u >= _NU:
                continue
            smem_off[u] = off

            def l2_loop(ci, acc, u=u, b0=b0, b1=b1):
                e = s_pack[pl.ds(ci * _L, _L)]
                gpos = iota + ci * _L
                valid = jnp.logical_and(gpos >= b0, gpos < b1)
                msk = jnp.logical_and(
                    valid, lax.shift_right_logical(e, 23) == u)
                plsc.store_compressed(u_pack.at[pl.ds(acc, _L)], e, mask=msk)
                return acc + _pc(msk)

            off = lax.fori_loop(c0b, c1b, l2_loop, off, unroll=False)
    smem_off[_NU] = off

    for jj in range(_STG // _L):
        bstage[pl.ds(jj * _L, _L)] = jnp.full((_L,), -1, jnp.int32)

    def scatter_staged():
        for jj in range(128 // _L):
            bvec[pl.ds(jj * _L, _L)] = bstage[pl.ds(jj * _L, _L)]
        pltpu.async_copy(
            pack2d.at[pl.ds(0, 128)],
            out.at[plsc.Indices(bvec, ignored_value=-1)],
            sem2,
        ).wait()

    def extract_chunks(start, end, parity, nrow):
        c0 = lax.shift_right_logical(start, 4)
        c1 = lax.shift_right_logical(end + _L - 1, 4)

        def chunk(ci, nrow):
            e = u_pack[pl.ds(ci * _L, _L)]
            gpos = iota + ci * _L
            msk = jnp.logical_and(gpos >= start, gpos < end)
            lane = lax.bitwise_and(e, 511)
            bb = lax.bitwise_and(lax.shift_right_logical(e, 9), 16383)
            pvec = jnp.broadcast_to(parity, (_L,))
            pos = nrow + plsc.cumsum(jnp.where(msk, 1, 0)) - 1
            for cc in range(_D):
                vals = plsc.load_gather(
                    blk, [pvec, jnp.full((_L,), cc, jnp.int32), lane])
                plsc.store_scatter(pack2d,
                                   [pos, jnp.full((_L,), cc, jnp.int32)],
                                   vals, mask=msk)
            plsc.store_scatter(bstage, [pos], bb, mask=msk)
            nrow = nrow + _pc(msk)

            do_flush = nrow >= 128

            @pl.when(do_flush)
            def _():
                scatter_staged()
                for r in range(_L):
                    for c4 in range(_D // _L):
                        pack2d[r, pl.ds(c4 * _L, _L)] = (
                            pack2d[128 + r, pl.ds(c4 * _L, _L)])
                bstage[pl.ds(0, _L)] = bstage[pl.ds(128, _L)]
                for jj in range(1, _STG // _L):
                    bstage[pl.ds(jj * _L, _L)] = jnp.full((_L,), -1, jnp.int32)

            return jnp.where(do_flush, nrow - 128, nrow)

        return lax.fori_loop(c0, c1, chunk, nrow, unroll=False)

    def group(u, nrow):
        parity = lax.bitwise_and(u, 1)
        pltpu.make_async_copy(quad_src(u), blk.at[parity], sem).wait()

        @pl.when(u < _NU - 2)
        def _():
            pltpu.async_copy(quad_src(u + 1),
                             blk.at[lax.bitwise_and(u + 1, 1)], sem)

        start = smem_off[u]
        end = smem_off[u + 1]
        return extract_chunks(start, end, parity, nrow)

    nrow = lax.fori_loop(0, _NU - 1, group, jnp.int32(0), unroll=False)

    t_start = smem_off[_NU - 1]
    t_end = smem_off[_NU]
    has_tail = t_end > t_start

    @pl.when(jnp.logical_and(has_tail, w == 0))
    def _():
        pltpu.sync_copy(quad_src(jnp.int32(_NU - 1)), blk.at[1])

    @pl.when(jnp.logical_and(has_tail, w == 1))
    def _():
        offl = pl.multiple_of((32 * (_NU - 1) + w) * 512, 128)
        pltpu.sync_copy(tableT.at[:, pl.ds(offl, 128)],
                        blk.at[1, :, pl.ds(0, 128)])

    nrow = extract_chunks(t_start, t_end, jnp.int32(1), nrow)

    for jj in range(_STG // _L):
        v = bstage[pl.ds(jj * _L, _L)]
        keep = (iota + jj * _L) < nrow
        bstage[pl.ds(jj * _L, _L)] = jnp.where(keep, v, -1)

    scatter_staged()


def _make_kernel():
    mesh = plsc.VectorSubcoreMesh(core_axis_name="c", subcore_axis_name="s")
    return functools.partial(
        pl.kernel,
        mesh=mesh,
        out_type=jax.ShapeDtypeStruct((_B, 128), jnp.float32),
        scratch_types=[
            pltpu.VMEM((2048,), jnp.int32),
            pltpu.VMEM((_B + _L * 2,), jnp.int32),
            pltpu.VMEM((_B + _L * 2,), jnp.int32),
            pltpu.VMEM((2, _D, 512), jnp.float32),
            pltpu.VMEM((_STG, 128), jnp.float32),
            pltpu.VMEM((_STG,), jnp.int32),
            pltpu.VMEM((128,), jnp.int32),
            pltpu.SMEM((_NU + 2,), jnp.int32),
            pltpu.SemaphoreType.DMA,
            pltpu.SemaphoreType.DMA,
        ],
        compiler_params=pltpu.CompilerParams(needs_layout_passes=False),
    )(_kernel_body)


def kernel(labels, embedding_table):
    kern = _make_kernel()
    lab = labels.astype(jnp.int32)
    tT = jnp.swapaxes(embedding_table, 0, 1)
    out128 = kern(tT, lab)
    return out128[:, :_D]


# speedup vs baseline: 1.4239x; 1.2412x over previous
import functools

import jax
import jax.numpy as jnp
from jax import lax
from jax.experimental import pallas as pl
from jax.experimental.pallas import tpu as pltpu
from jax.experimental.pallas import tpu_sc as plsc

_B = 16384
_D = 64
_V = 1000000
_L = 16
_NC = 2
_NS = 16
_NW = _NC * _NS
_NP = 1954
_NU = 62
_STG = 144


def _pc(mask):
    return plsc.all_reduce_population_count(mask)[0]


def _kernel_body(tableT, labels, out,
                 lblstage, u_pack, s_pack, blk, pack2d, bstage, bvec,
                 smem_off, sem, sem2):
    w = lax.axis_index("s") * _NC + lax.axis_index("c")
    iota = lax.iota(jnp.int32, _L)

    def quad_src(u):
        offl = pl.multiple_of((32 * u + w) * 512, 128)
        return tableT.at[:, pl.ds(offl, 512)]

    pltpu.async_copy(quad_src(jnp.int32(0)), blk.at[0], sem)
    pltpu.async_copy(quad_src(jnp.int32(1)), blk.at[1], sem)

    def stage_loop(st, carry):
        pltpu.sync_copy(
            labels.at[pl.ds(pl.multiple_of(st * 2048, 2048), 2048)], lblstage)

        def chunk_loop(ci, n):
            lb = lblstage[pl.ds(ci * _L, _L)]
            owner = lax.bitwise_and(lax.shift_right_logical(lb, 9), 31)
            mine = owner == w
            bpos = iota + (st * 2048 + ci * _L)
            entry = lax.bitwise_or(
                lax.shift_left(lax.shift_right_logical(lb, 14), 23),
                lax.bitwise_or(lax.shift_left(bpos, 9),
                               lax.bitwise_and(lb, 511)))
            plsc.store_compressed(u_pack.at[pl.ds(n, _L)], entry, mask=mine)
            return n + _pc(mine)

        return lax.fori_loop(0, 2048 // _L, chunk_loop, carry, unroll=4)

    n_mine = lax.fori_loop(0, _B // 2048, stage_loop, jnp.int32(0),
                           unroll=False)

    nch = lax.shift_right_logical(n_mine + _L - 1, 4)
    off = jnp.int32(0)
    lvl1 = [jnp.int32(0)] * 9
    for s1 in range(8):
        lvl1[s1] = off

        def l1_loop(ci, acc, s1=s1):
            e = u_pack[pl.ds(ci * _L, _L)]
            valid = (iota + ci * _L) < n_mine
            msk = jnp.logical_and(valid, lax.shift_right_logical(e, 26) == s1)
            plsc.store_compressed(s_pack.at[pl.ds(acc, _L)], e, mask=msk)
            return acc + _pc(msk)

        off = lax.fori_loop(0, nch, l1_loop, off, unroll=False)
    lvl1[8] = off

    off = jnp.int32(0)
    for s1 in range(8):
        b0 = lvl1[s1]
        b1 = lvl1[s1 + 1]
        c0b = lax.shift_right_logical(b0, 4)
        c1b = lax.shift_right_logical(b1 + _L - 1, 4)
        for s2 in range(8):
            u = s1 * 8 + s2
            if u >= _NU:
                continue
            smem_off[u] = off

            def l2_loop(ci, acc, u=u, b0=b0, b1=b1):
                e = s_pack[pl.ds(ci * _L, _L)]
                gpos = iota + ci * _L
                valid = jnp.logical_and(gpos >= b0, gpos < b1)
                msk = jnp.logical_and(
                    valid, lax.shift_right_logical(e, 23) == u)
                plsc.store_compressed(u_pack.at[pl.ds(acc, _L)], e, mask=msk)
                return acc + _pc(msk)

            off = lax.fori_loop(c0b, c1b, l2_loop, off, unroll=False)
    smem_off[_NU] = off

    for jj in range(_STG // _L):
        bstage[pl.ds(jj * _L, _L)] = jnp.full((_L,), -1, jnp.int32)

    def scatter_staged():
        for jj in range(128 // _L):
            bvec[pl.ds(jj * _L, _L)] = bstage[pl.ds(jj * _L, _L)]
        pltpu.async_copy(
            pack2d.at[pl.ds(0, 128)],
            out.at[plsc.Indices(bvec, ignored_value=-1)],
            sem2,
        ).wait()

    def extract_chunks(start, end, parity, nrow):
        c0 = lax.shift_right_logical(start, 4)
        c1 = lax.shift_right_logical(end + _L - 1, 4)

        def chunk(ci, nrow):
            e = u_pack[pl.ds(ci * _L, _L)]
            gpos = iota + ci * _L
            msk = jnp.logical_and(gpos >= start, gpos < end)
            lane = lax.bitwise_and(e, 511)
            bb = lax.bitwise_and(lax.shift_right_logical(e, 9), 16383)
            pvec = jnp.broadcast_to(parity, (_L,))
            pos = nrow + plsc.cumsum(jnp.where(msk, 1, 0)) - 1
            for cc in range(_D):
                vals = plsc.load_gather(
                    blk, [pvec, jnp.full((_L,), cc, jnp.int32), lane])
                plsc.store_scatter(pack2d,
                                   [pos, jnp.full((_L,), cc, jnp.int32)],
                                   vals, mask=msk)
            plsc.store_scatter(bstage, [pos], bb, mask=msk)
            nrow = nrow + _pc(msk)

            do_flush = nrow >= 128

            @pl.when(do_flush)
            def _():
                scatter_staged()
                for r in range(_L):
                    for c4 in range(_D // _L):
                        pack2d[r, pl.ds(c4 * _L, _L)] = (
                            pack2d[128 + r, pl.ds(c4 * _L, _L)])
                bstage[pl.ds(0, _L)] = bstage[pl.ds(128, _L)]
                for jj in range(1, _STG // _L):
                    bstage[pl.ds(jj * _L, _L)] = jnp.full((_L,), -1, jnp.int32)

            return jnp.where(do_flush, nrow - 128, nrow)

        return lax.fori_loop(c0, c1, chunk, nrow, unroll=False)

    def group(u, nrow):
        parity = lax.bitwise_and(u, 1)
        pltpu.make_async_copy(quad_src(u), blk.at[parity], sem).wait()

        start = smem_off[u]
        end = smem_off[u + 1]
        nrow = extract_chunks(start, end, parity, nrow)

        @pl.when(u < _NU - 3)
        def _():
            pltpu.async_copy(quad_src(u + 2), blk.at[parity], sem)

        return nrow

    nrow = lax.fori_loop(0, _NU - 1, group, jnp.int32(0), unroll=False)

    t_start = smem_off[_NU - 1]
    t_end = smem_off[_NU]
    has_tail = t_end > t_start

    @pl.when(jnp.logical_and(has_tail, w == 0))
    def _():
        pltpu.sync_copy(quad_src(jnp.int32(_NU - 1)), blk.at[1])

    @pl.when(jnp.logical_and(has_tail, w == 1))
    def _():
        offl = pl.multiple_of((32 * (_NU - 1) + w) * 512, 128)
        pltpu.sync_copy(tableT.at[:, pl.ds(offl, 128)],
                        blk.at[1, :, pl.ds(0, 128)])

    nrow = extract_chunks(t_start, t_end, jnp.int32(1), nrow)

    for jj in range(_STG // _L):
        v = bstage[pl.ds(jj * _L, _L)]
        keep = (iota + jj * _L) < nrow
        bstage[pl.ds(jj * _L, _L)] = jnp.where(keep, v, -1)

    scatter_staged()


def _make_kernel():
    mesh = plsc.VectorSubcoreMesh(core_axis_name="c", subcore_axis_name="s")
    return functools.partial(
        pl.kernel,
        mesh=mesh,
        out_type=jax.ShapeDtypeStruct((_B, 128), jnp.float32),
        scratch_types=[
            pltpu.VMEM((2048,), jnp.int32),
            pltpu.VMEM((_B + _L * 2,), jnp.int32),
            pltpu.VMEM((_B + _L * 2,), jnp.int32),
            pltpu.VMEM((2, _D, 512), jnp.float32),
            pltpu.VMEM((_STG, 128), jnp.float32),
            pltpu.VMEM((_STG,), jnp.int32),
            pltpu.VMEM((128,), jnp.int32),
            pltpu.SMEM((_NU + 2,), jnp.int32),
            pltpu.SemaphoreType.DMA,
            pltpu.SemaphoreType.DMA,
        ],
        compiler_params=pltpu.CompilerParams(needs_layout_passes=False),
    )(_kernel_body)


def kernel(labels, embedding_table):
    kern = _make_kernel()
    lab = labels.astype(jnp.int32)
    tT = jnp.swapaxes(embedding_table, 0, 1)
    out128 = kern(tT, lab)
    return out128[:, :_D]
